# Initial kernel scaffold; baseline (speedup 1.0000x reference)
import jax
import jax.numpy as jnp
from jax.experimental import pallas as pl


def kernel(x, edge_index, W_embed, b_embed, Wc0, bc0, Wc1, bc1, W0, b0, W1, b1, W2, b2):
    raise NotImplementedError("write your pallas kernel here")


# speedup vs baseline: 25.3096x; 25.3096x over previous
import functools

import jax
import jax.numpy as jnp
from jax import lax
from jax.experimental import pallas as pl
from jax.experimental.pallas import tpu as pltpu
from jax.experimental.pallas import tpu_sc as plsc

N = 10000
HID = 128
E = 320000
NC = 2
NS = 16
CH = 125
RPT = E // (NC * NS * CH)
ZCH = 200
NZCH = N // ZCH
ZITER = (NZCH + NS - 1) // NS

_MESH = plsc.VectorSubcoreMesh(
    core_axis_name="c", subcore_axis_name="s", num_cores=NC, num_subcores=NS)
_SC_PARAMS = pltpu.CompilerParams(use_tc_tiling_on_sc=False)


@functools.partial(
    pl.kernel,
    out_type=jax.ShapeDtypeStruct((NC, N, 16), jnp.float32),
    mesh=_MESH,
    scratch_types=[
        pltpu.VMEM((RPT, CH), jnp.int32),
        pltpu.VMEM((CH, 16), jnp.float32),
        pltpu.VMEM((ZCH, 16), jnp.float32),
        pltpu.VMEM_SHARED((N, 16), jnp.float32),
    ],
    compiler_params=_SC_PARAMS,
)
def _sc_deg(dst2d, ones16, zeros16, out, dbuf, obuf, zbuf, deg_sp):
    c = lax.axis_index("c")
    s = lax.axis_index("s")
    pltpu.sync_copy(ones16, obuf)
    pltpu.sync_copy(zeros16, zbuf)
    for k in range(ZITER):
        j = s + NS * k

        @pl.when(j < NZCH)
        def _():
            pltpu.sync_copy(zbuf, deg_sp.at[pl.ds(pl.multiple_of(j * ZCH, 8), ZCH)])

    base_r = pl.multiple_of((c * NS + s) * RPT, 8)
    pltpu.sync_copy(dst2d.at[pl.ds(base_r, RPT)], dbuf)
    plsc.subcore_barrier()

    def chunk(j, carry):
        pltpu.sync_copy(obuf, deg_sp.at[dbuf.at[j]], add=True)
        return carry

    lax.fori_loop(0, RPT, chunk, 0)
    plsc.subcore_barrier()
    for k in range(ZITER):
        j = s + NS * k

        @pl.when(j < NZCH)
        def _():
            off = pl.multiple_of(j * ZCH, 8)
            pltpu.sync_copy(deg_sp.at[pl.ds(off, ZCH)], out.at[c, pl.ds(off, ZCH)])


HHID = HID // 2
RPT_F = E // (NS * CH)


@functools.partial(
    pl.kernel,
    out_type=jax.ShapeDtypeStruct((NC, N, HHID), jnp.float32),
    mesh=_MESH,
    scratch_types=[
        pltpu.VMEM((RPT_F, CH), jnp.int32),
        pltpu.VMEM((RPT_F, CH), jnp.int32),
        pltpu.VMEM((CH, HHID), jnp.float32),
        pltpu.VMEM((CH, HHID), jnp.float32),
        pltpu.VMEM((ZCH, HHID), jnp.float32),
        pltpu.VMEM_SHARED((N, HHID), jnp.float32),
        pltpu.SemaphoreType.DMA,
        pltpu.SemaphoreType.DMA,
    ],
    compiler_params=_SC_PARAMS,
)
def _sc_prop(src2d, dst2d, mlo, mhi, zeros64, out, sbuf, dbuf, rows0, rows1,
             zbuf, acc_sp, gsem0, gsem1):
    c = lax.axis_index("c")
    s = lax.axis_index("s")
    pltpu.sync_copy(zeros64, zbuf)
    for k in range(ZITER):
        j = s + NS * k

        @pl.when(j < NZCH)
        def _():
            pltpu.sync_copy(zbuf, acc_sp.at[pl.ds(pl.multiple_of(j * ZCH, 8), ZCH)])

    base_r = pl.multiple_of(s * RPT_F, 8)
    pltpu.sync_copy(src2d.at[pl.ds(base_r, RPT_F)], sbuf)
    pltpu.sync_copy(dst2d.at[pl.ds(base_r, RPT_F)], dbuf)
    plsc.subcore_barrier()

    def run(table):
        pltpu.async_copy(table.at[sbuf.at[0]], rows0, gsem0)

        def chunk2(i, carry):
            j0 = 2 * i
            pltpu.async_copy(table.at[sbuf.at[j0 + 1]], rows1, gsem1)
            pltpu.make_async_copy(table.at[sbuf.at[j0]], rows0, gsem0).wait()
            pltpu.sync_copy(rows0, acc_sp.at[dbuf.at[j0]], add=True)

            @pl.when(j0 + 2 < RPT_F)
            def _():
                pltpu.async_copy(table.at[sbuf.at[j0 + 2]], rows0, gsem0)

            pltpu.make_async_copy(table.at[sbuf.at[j0 + 1]], rows1, gsem1).wait()
            pltpu.sync_copy(rows1, acc_sp.at[dbuf.at[j0 + 1]], add=True)
            return carry

        lax.fori_loop(0, RPT_F // 2, chunk2, 0)

    @pl.when(c == 0)
    def _():
        run(mlo)

    @pl.when(c == 1)
    def _():
        run(mhi)

    plsc.subcore_barrier()
    for k in range(ZITER):
        j = s + NS * k

        @pl.when(j < NZCH)
        def _():
            off = pl.multiple_of(j * ZCH, 8)
            pltpu.sync_copy(acc_sp.at[pl.ds(off, ZCH)], out.at[c, pl.ds(off, ZCH)])


_R = 1000
_G = N // _R

def _full(shape):
    return pl.BlockSpec(shape, lambda i: (0,) * len(shape))

def _rows(width):
    return pl.BlockSpec((_R, width), lambda i: (i, 0))

_DEG_SPEC = pl.BlockSpec((NC, _R, 16), lambda i: (0, i, 0))
_ACC_SPEC = pl.BlockSpec((NC, _R, HHID), lambda i: (0, i, 0))


def _dis(deg_ref):
    d = deg_ref[0, :, 0:1] + deg_ref[1, :, 0:1] + 1.0
    return lax.rsqrt(d)


def _embed_body(x_ref, wet_ref, be_ref, wc0t_ref, h_ref, xw0_ref):
    h = jnp.dot(x_ref[...], wet_ref[...], preferred_element_type=jnp.float32)
    h = h + be_ref[...]
    h_ref[...] = h
    xw0_ref[...] = jnp.dot(h, wc0t_ref[...], preferred_element_type=jnp.float32)


_tc_embed = pl.pallas_call(
    _embed_body,
    grid=(_G,),
    in_specs=[_rows(HID), _full((HID, HID)), _full((1, HID)), _full((HID, HID))],
    out_specs=[_rows(HID), _rows(HID)],
    out_shape=[jax.ShapeDtypeStruct((N, HID), jnp.float32)] * 2,
)


def _mkm0_body(deg_ref, xw0_ref, mlo_ref, mhi_ref):
    m0 = xw0_ref[...] * _dis(deg_ref)
    mlo_ref[...] = m0[:, :HHID]
    mhi_ref[...] = m0[:, HHID:]


_tc_mkm0 = pl.pallas_call(
    _mkm0_body,
    grid=(_G,),
    in_specs=[_DEG_SPEC, _rows(HID)],
    out_specs=[_rows(HHID), _rows(HHID)],
    out_shape=[jax.ShapeDtypeStruct((N, HHID), jnp.float32)] * 2,
)


def _layer_body(deg_ref, acc_ref, mlo_ref, mhi_ref, h_ref, bc_ref, wct_ref,
                h1_ref, m1lo_ref, m1hi_ref):
    dis = _dis(deg_ref)
    m = jnp.concatenate([mlo_ref[...], mhi_ref[...]], axis=1)
    acc = jnp.concatenate([acc_ref[0, :, :], acc_ref[1, :, :]], axis=1)
    conv = dis * (acc + m) + bc_ref[...]
    h1 = jnp.maximum(conv, 0.0) + h_ref[...]
    h1_ref[...] = h1
    m1 = dis * jnp.dot(h1, wct_ref[...], preferred_element_type=jnp.float32)
    m1lo_ref[...] = m1[:, :HHID]
    m1hi_ref[...] = m1[:, HHID:]


_tc_layer = pl.pallas_call(
    _layer_body,
    grid=(_G,),
    in_specs=[_DEG_SPEC, _ACC_SPEC, _rows(HHID), _rows(HHID), _rows(HID),
              _full((1, HID)), _full((HID, HID))],
    out_specs=[_rows(HID), _rows(HHID), _rows(HHID)],
    out_shape=[jax.ShapeDtypeStruct((N, HID), jnp.float32),
               jax.ShapeDtypeStruct((N, HHID), jnp.float32),
               jax.ShapeDtypeStruct((N, HHID), jnp.float32)],
)


def _final_body(deg_ref, acc_ref, mlo_ref, mhi_ref, h_ref, bc_ref,
                w0t_ref, b0_ref, w1t_ref, b1_ref, w2t_ref, b2_ref, out_ref):
    dis = _dis(deg_ref)
    m = jnp.concatenate([mlo_ref[...], mhi_ref[...]], axis=1)
    acc = jnp.concatenate([acc_ref[0, :, :], acc_ref[1, :, :]], axis=1)
    conv = dis * (acc + m) + bc_ref[...]
    h2 = jnp.maximum(conv, 0.0) + h_ref[...]
    t = jnp.maximum(jnp.dot(h2, w0t_ref[...], preferred_element_type=jnp.float32)
                    + b0_ref[...], 0.0)
    t = jnp.maximum(jnp.dot(t, w1t_ref[...], preferred_element_type=jnp.float32)
                    + b1_ref[...], 0.0)
    out_ref[...] = (jnp.dot(t, w2t_ref[...], preferred_element_type=jnp.float32)
                    + b2_ref[...])


_tc_final = pl.pallas_call(
    _final_body,
    grid=(_G,),
    in_specs=[_DEG_SPEC, _ACC_SPEC, _rows(HHID), _rows(HHID), _rows(HID),
              _full((1, HID)),
              _full((HID, 64)), _full((1, 64)), _full((64, 32)), _full((1, 32)),
              _full((32, 16)), _full((1, 16))],
    out_specs=pl.BlockSpec((_R, 16), lambda i: (i, 0)),
    out_shape=jax.ShapeDtypeStruct((N, 16), jnp.float32),
)


def kernel(x, edge_index, W_embed, b_embed, Wc0, bc0, Wc1, bc1,
           W0, b0, W1, b1, W2, b2):
    ei = edge_index.astype(jnp.int32)
    src2d = ei[0].reshape(E // CH, CH)
    dst2d = ei[1].reshape(E // CH, CH)
    ones16 = jnp.ones((CH, 16), jnp.float32)
    zeros16 = jnp.zeros((ZCH, 16), jnp.float32)
    zeros64 = jnp.zeros((ZCH, HHID), jnp.float32)

    deg2 = _sc_deg(dst2d, ones16, zeros16)
    h, xw0 = _tc_embed(x, W_embed.T, b_embed.reshape(1, HID), Wc0.T)
    m0lo, m0hi = _tc_mkm0(deg2, xw0)
    acc0 = _sc_prop(src2d, dst2d, m0lo, m0hi, zeros64)
    h1, m1lo, m1hi = _tc_layer(deg2, acc0, m0lo, m0hi, h,
                               bc0.reshape(1, HID), Wc1.T)
    acc1 = _sc_prop(src2d, dst2d, m1lo, m1hi, zeros64)
    return _tc_final(deg2, acc1, m1lo, m1hi, h1, bc1.reshape(1, HID),
                     W0.T, b0.reshape(1, 64), W1.T, b1.reshape(1, 32),
                     W2.T, b2.reshape(1, 16))


# speedup vs baseline: 27.5551x; 1.0887x over previous
import functools

import jax
import jax.numpy as jnp
from jax import lax
from jax.experimental import pallas as pl
from jax.experimental.pallas import tpu as pltpu
from jax.experimental.pallas import tpu_sc as plsc

N = 10000
HID = 128
E = 320000
NC = 2
NS = 16
CH = 125
RPT = E // (NC * NS * CH)
ZCH = 200
NZCH = N // ZCH
ZITER = (NZCH + NS - 1) // NS

_MESH = plsc.VectorSubcoreMesh(
    core_axis_name="c", subcore_axis_name="s", num_cores=NC, num_subcores=NS)
_SC_PARAMS = pltpu.CompilerParams(use_tc_tiling_on_sc=False)


@functools.partial(
    pl.kernel,
    out_type=jax.ShapeDtypeStruct((NC, N, 16), jnp.float32),
    mesh=_MESH,
    scratch_types=[
        pltpu.VMEM((RPT, CH), jnp.int32),
        pltpu.VMEM((CH, 16), jnp.float32),
        pltpu.VMEM((ZCH, 16), jnp.float32),
        pltpu.VMEM_SHARED((N, 16), jnp.float32),
    ],
    compiler_params=_SC_PARAMS,
)
def _sc_deg(dst2d, ones16, zeros16, out, dbuf, obuf, zbuf, deg_sp):
    c = lax.axis_index("c")
    s = lax.axis_index("s")
    pltpu.sync_copy(ones16, obuf)
    pltpu.sync_copy(zeros16, zbuf)
    for k in range(ZITER):
        j = s + NS * k

        @pl.when(j < NZCH)
        def _():
            pltpu.sync_copy(zbuf, deg_sp.at[pl.ds(pl.multiple_of(j * ZCH, 8), ZCH)])

    base_r = pl.multiple_of((c * NS + s) * RPT, 8)
    pltpu.sync_copy(dst2d.at[pl.ds(base_r, RPT)], dbuf)
    plsc.subcore_barrier()

    def chunk(j, carry):
        pltpu.sync_copy(obuf, deg_sp.at[dbuf.at[j]], add=True)
        return carry

    lax.fori_loop(0, RPT, chunk, 0)
    plsc.subcore_barrier()
    for k in range(ZITER):
        j = s + NS * k

        @pl.when(j < NZCH)
        def _():
            off = pl.multiple_of(j * ZCH, 8)
            pltpu.sync_copy(deg_sp.at[pl.ds(off, ZCH)], out.at[c, pl.ds(off, ZCH)])


HHID = HID // 2
RPT_F = E // (NS * CH)
K = 4
NSEG = 2
RSEG = RPT_F // NSEG
NRH = RSEG // K
STRIPE = N // NS
NZC = STRIPE // CH


@functools.partial(
    pl.kernel,
    out_type=jax.ShapeDtypeStruct((NC, N, HHID), jnp.float32),
    mesh=_MESH,
    scratch_types=[
        pltpu.VMEM((RSEG, CH), jnp.int32),
        pltpu.VMEM((RSEG, CH), jnp.int32),
        [pltpu.VMEM((CH, HHID), jnp.float32)] * (2 * K),
        pltpu.VMEM_SHARED((N, HHID), jnp.float32),
        pltpu.SemaphoreType.DMA,
        pltpu.SemaphoreType.DMA,
        pltpu.SemaphoreType.DMA,
        pltpu.SemaphoreType.DMA,
    ],
    compiler_params=_SC_PARAMS,
)
def _sc_prop(src2d, dst2d, mlo, mhi, zeros64, out, sbuf, dbuf, bufs,
             acc_sp, gsa, gsb, ssa, ssb):
    c = lax.axis_index("c")
    s = lax.axis_index("s")
    buf_a, buf_b = bufs[:K], bufs[K:]
    pltpu.sync_copy(zeros64, buf_a[0])
    for k in range(NZC):
        pltpu.sync_copy(buf_a[0], acc_sp.at[pl.ds(s * STRIPE + k * CH, CH)])

    def run(table):
        def fire_g(r, bufset, sem):
            base = r * K
            for k in range(K):
                pltpu.async_copy(table.at[sbuf.at[base + k]], bufset[k], sem)

        def drain_g(bufset, sem):
            for k in range(K):
                pltpu.make_async_copy(table.at[sbuf.at[0]], bufset[k], sem).wait()

        def fire_s(r, bufset, sem):
            base = r * K
            for k in range(K):
                pltpu.async_copy(bufset[k], acc_sp.at[dbuf.at[base + k]], sem,
                                 add=True)

        def drain_s(bufset, sem):
            for k in range(K):
                pltpu.make_async_copy(bufset[k], acc_sp.at[dbuf.at[0]],
                                      sem).wait()

        def body(i, carry):
            r0 = 2 * i
            drain_g(buf_a, gsa)
            fire_s(r0, buf_a, ssa)
            drain_s(buf_a, ssa)

            @pl.when(r0 + 2 < NRH)
            def _():
                fire_g(r0 + 2, buf_a, gsa)

            drain_g(buf_b, gsb)
            fire_s(r0 + 1, buf_b, ssb)
            drain_s(buf_b, ssb)

            @pl.when(r0 + 3 < NRH)
            def _():
                fire_g(r0 + 3, buf_b, gsb)

            return carry

        for seg in range(NSEG):
            base_r = pl.multiple_of(s * RPT_F + seg * RSEG, 8)
            pltpu.sync_copy(src2d.at[pl.ds(base_r, RSEG)], sbuf)
            pltpu.sync_copy(dst2d.at[pl.ds(base_r, RSEG)], dbuf)
            if seg == 0:
                plsc.subcore_barrier()
            fire_g(0, buf_a, gsa)
            fire_g(1, buf_b, gsb)
            lax.fori_loop(0, NRH // 2, body, 0)

    @pl.when(c == 0)
    def _():
        run(mlo)

    @pl.when(c == 1)
    def _():
        run(mhi)

    plsc.subcore_barrier()
    for k in range(NZC):
        off = s * STRIPE + k * CH
        pltpu.sync_copy(acc_sp.at[pl.ds(off, CH)], out.at[c, pl.ds(off, CH)])


_R = 1000
_G = N // _R

def _full(shape):
    return pl.BlockSpec(shape, lambda i: (0,) * len(shape))

def _rows(width):
    return pl.BlockSpec((_R, width), lambda i: (i, 0))

_DEG_SPEC = pl.BlockSpec((NC, _R, 16), lambda i: (0, i, 0))
_ACC_SPEC = pl.BlockSpec((NC, _R, HHID), lambda i: (0, i, 0))


def _dis(deg_ref):
    d = deg_ref[0, :, 0:1] + deg_ref[1, :, 0:1] + 1.0
    return lax.rsqrt(d)


def _embed_body(x_ref, wet_ref, be_ref, wc0t_ref, h_ref, xw0_ref):
    h = jnp.dot(x_ref[...], wet_ref[...], preferred_element_type=jnp.float32)
    h = h + be_ref[...]
    h_ref[...] = h
    xw0_ref[...] = jnp.dot(h, wc0t_ref[...], preferred_element_type=jnp.float32)


_tc_embed = pl.pallas_call(
    _embed_body,
    grid=(_G,),
    in_specs=[_rows(HID), _full((HID, HID)), _full((1, HID)), _full((HID, HID))],
    out_specs=[_rows(HID), _rows(HID)],
    out_shape=[jax.ShapeDtypeStruct((N, HID), jnp.float32)] * 2,
)


def _mkm0_body(deg_ref, xw0_ref, mlo_ref, mhi_ref):
    m0 = xw0_ref[...] * _dis(deg_ref)
    mlo_ref[...] = m0[:, :HHID]
    mhi_ref[...] = m0[:, HHID:]


_tc_mkm0 = pl.pallas_call(
    _mkm0_body,
    grid=(_G,),
    in_specs=[_DEG_SPEC, _rows(HID)],
    out_specs=[_rows(HHID), _rows(HHID)],
    out_shape=[jax.ShapeDtypeStruct((N, HHID), jnp.float32)] * 2,
)


def _layer_body(deg_ref, acc_ref, mlo_ref, mhi_ref, h_ref, bc_ref, wct_ref,
                h1_ref, m1lo_ref, m1hi_ref):
    dis = _dis(deg_ref)
    m = jnp.concatenate([mlo_ref[...], mhi_ref[...]], axis=1)
    acc = jnp.concatenate([acc_ref[0, :, :], acc_ref[1, :, :]], axis=1)
    conv = dis * (acc + m) + bc_ref[...]
    h1 = jnp.maximum(conv, 0.0) + h_ref[...]
    h1_ref[...] = h1
    m1 = dis * jnp.dot(h1, wct_ref[...], preferred_element_type=jnp.float32)
    m1lo_ref[...] = m1[:, :HHID]
    m1hi_ref[...] = m1[:, HHID:]


_tc_layer = pl.pallas_call(
    _layer_body,
    grid=(_G,),
    in_specs=[_DEG_SPEC, _ACC_SPEC, _rows(HHID), _rows(HHID), _rows(HID),
              _full((1, HID)), _full((HID, HID))],
    out_specs=[_rows(HID), _rows(HHID), _rows(HHID)],
    out_shape=[jax.ShapeDtypeStruct((N, HID), jnp.float32),
               jax.ShapeDtypeStruct((N, HHID), jnp.float32),
               jax.ShapeDtypeStruct((N, HHID), jnp.float32)],
)


def _final_body(deg_ref, acc_ref, mlo_ref, mhi_ref, h_ref, bc_ref,
                w0t_ref, b0_ref, w1t_ref, b1_ref, w2t_ref, b2_ref, out_ref):
    dis = _dis(deg_ref)
    m = jnp.concatenate([mlo_ref[...], mhi_ref[...]], axis=1)
    acc = jnp.concatenate([acc_ref[0, :, :], acc_ref[1, :, :]], axis=1)
    conv = dis * (acc + m) + bc_ref[...]
    h2 = jnp.maximum(conv, 0.0) + h_ref[...]
    t = jnp.maximum(jnp.dot(h2, w0t_ref[...], preferred_element_type=jnp.float32)
                    + b0_ref[...], 0.0)
    t = jnp.maximum(jnp.dot(t, w1t_ref[...], preferred_element_type=jnp.float32)
                    + b1_ref[...], 0.0)
    out_ref[...] = (jnp.dot(t, w2t_ref[...], preferred_element_type=jnp.float32)
                    + b2_ref[...])


_tc_final = pl.pallas_call(
    _final_body,
    grid=(_G,),
    in_specs=[_DEG_SPEC, _ACC_SPEC, _rows(HHID), _rows(HHID), _rows(HID),
              _full((1, HID)),
              _full((HID, 64)), _full((1, 64)), _full((64, 32)), _full((1, 32)),
              _full((32, 16)), _full((1, 16))],
    out_specs=pl.BlockSpec((_R, 16), lambda i: (i, 0)),
    out_shape=jax.ShapeDtypeStruct((N, 16), jnp.float32),
)


def kernel(x, edge_index, W_embed, b_embed, Wc0, bc0, Wc1, bc1,
           W0, b0, W1, b1, W2, b2):
    ei = edge_index.astype(jnp.int32)
    src2d = ei[0].reshape(E // CH, CH)
    dst2d = ei[1].reshape(E // CH, CH)
    ones16 = jnp.ones((CH, 16), jnp.float32)
    zeros16 = jnp.zeros((ZCH, 16), jnp.float32)
    zeros64 = jnp.zeros((CH, HHID), jnp.float32)

    deg2 = _sc_deg(dst2d, ones16, zeros16)
    h, xw0 = _tc_embed(x, W_embed.T, b_embed.reshape(1, HID), Wc0.T)
    m0lo, m0hi = _tc_mkm0(deg2, xw0)
    acc0 = _sc_prop(src2d, dst2d, m0lo, m0hi, zeros64)
    h1, m1lo, m1hi = _tc_layer(deg2, acc0, m0lo, m0hi, h,
                               bc0.reshape(1, HID), Wc1.T)
    acc1 = _sc_prop(src2d, dst2d, m1lo, m1hi, zeros64)
    return _tc_final(deg2, acc1, m1lo, m1hi, h1, bc1.reshape(1, HID),
                     W0.T, b0.reshape(1, 64), W1.T, b1.reshape(1, 32),
                     W2.T, b2.reshape(1, 16))
